# baseline (speedup 1.0000x reference)
import functools

import jax
import jax.numpy as jnp
from jax import lax
from jax.experimental import pallas as pl
from jax.experimental.pallas import tpu as pltpu
from jax.experimental.pallas import tpu_sc as plsc

NUM_ARMS = 1000000
D = 32
B = 4096
S = 50
N = B * S

NC = 2
NS = 16
NW = NC * NS
CHUNK = B // NW
L = 16


def _sc_gather_body(action_hbm, table4_hbm, out_hbm, idx_v, idx4_v,
                    buf0, buf1, st0, st1, sem0, sem1):
  wid = lax.axis_index("s") * NC + lax.axis_index("c")
  bbase = wid * CHUNK
  pltpu.sync_copy(action_hbm.at[:, pl.ds(bbase, CHUNK)], idx_v)

  def mk_idx4(i, _):
    def row(s, j):
      sl = pl.ds(j * L, L)
      idx4_v[s, sl] = lax.shift_right_logical(idx_v[s, sl], 2)
    for j in range(CHUNK // L):
      row(i, j)
    return 0

  lax.fori_loop(0, S, mk_idx4, 0)

  bufs = (buf0, buf1)
  stages = (st0, st1)
  sems = (sem0, sem1)
  kvecs = [lax.iota(jnp.int32, L) + L * g for g in range(CHUNK // L)]

  def extract(s, buf, stage):
    for g in range(CHUNK // L):
      sl = pl.ds(g * L, L)
      offv = lax.shift_left(jnp.bitwise_and(idx_v[s, sl], 3), 5)
      for d in range(D):
        stage[d, sl] = plsc.load_gather(buf, [kvecs[g], offv + d])

  pltpu.async_copy(table4_hbm.at[idx4_v.at[0]], buf0, sem0)
  pltpu.async_copy(table4_hbm.at[idx4_v.at[1]], buf1, sem1)

  def step(i, _):
    base = i * 2
    for b in range(2):
      s = base + b
      pltpu.make_async_copy(table4_hbm.at[idx4_v.at[s]], bufs[b],
                            sems[b]).wait()
      extract(s, bufs[b], stages[b])
      pltpu.sync_copy(stages[b], out_hbm.at[s, :, pl.ds(bbase, CHUNK)])
      @pl.when(s + 2 < S)
      def _():
        pltpu.async_copy(table4_hbm.at[idx4_v.at[s + 2]], bufs[b], sems[b])
    return 0

  lax.fori_loop(0, S // 2, step, 0)


_sc_gather = functools.partial(
    pl.kernel,
    out_type=jax.ShapeDtypeStruct((S, D, B), jnp.float32),
    mesh=plsc.VectorSubcoreMesh(core_axis_name="c", subcore_axis_name="s"),
    scratch_types=[
        pltpu.VMEM((S, CHUNK), jnp.int32),
        pltpu.VMEM((S, CHUNK), jnp.int32),
        pltpu.VMEM((CHUNK, 4 * D), jnp.float32),
        pltpu.VMEM((CHUNK, 4 * D), jnp.float32),
        pltpu.VMEM((D, CHUNK), jnp.float32),
        pltpu.VMEM((D, CHUNK), jnp.float32),
        pltpu.SemaphoreType.DMA,
        pltpu.SemaphoreType.DMA,
    ],
    compiler_params=pltpu.CompilerParams(use_tc_tiling_on_sc=True,
                                         needs_layout_passes=False),
)(_sc_gather_body)


def _tc_assemble_body(state_ref, reward_ref, act_ref, wo_ref, bo_ref,
                      wr_ref, br_ref, out_ref):
  wo = jnp.transpose(wo_ref[...])
  bo = jnp.transpose(bo_ref[...])
  wr = jnp.transpose(wr_ref[...])
  br = jnp.transpose(br_ref[...])
  st = state_ref[0]
  rw = reward_ref[0]
  out_ref[0] = wo * st + bo
  out_ref[1] = act_ref[0]
  out_ref[2] = wr * rw + br


TBB = 1024


def _tc_assemble(state_t, reward_t, act_t, W_obs, b_obs, W_rew, b_rew):
  grid = (S, B // TBB)
  return pl.pallas_call(
      _tc_assemble_body,
      grid=grid,
      in_specs=[
          pl.BlockSpec((1, 1, TBB), lambda s, j: (s, 0, j)),
          pl.BlockSpec((1, 1, TBB), lambda s, j: (s, 0, j)),
          pl.BlockSpec((1, D, TBB), lambda s, j: (s, 0, j)),
          pl.BlockSpec((1, D), lambda s, j: (0, 0)),
          pl.BlockSpec((1, D), lambda s, j: (0, 0)),
          pl.BlockSpec((1, D), lambda s, j: (0, 0)),
          pl.BlockSpec((1, D), lambda s, j: (0, 0)),
      ],
      out_specs=pl.BlockSpec((3, D, TBB), lambda s, j: (s, 0, j)),
      out_shape=jax.ShapeDtypeStruct((3 * S, D, B), jnp.float32),
  )(state_t, reward_t, act_t, W_obs, b_obs, W_rew, b_rew)


@jax.jit
def kernel(state, action, reward, W_obs, b_obs, emb_table, W_rew, b_rew):
  action_t = action.astype(jnp.int32).T
  state_t = state.transpose(1, 2, 0)
  reward_t = reward.T.reshape(S, 1, B)
  table4 = emb_table.reshape(NUM_ARMS // 4, 4 * D)
  act_t = _sc_gather(action_t, table4)
  out_t = _tc_assemble(
      state_t,
      reward_t,
      act_t,
      W_obs,
      b_obs.reshape(1, D),
      W_rew,
      b_rew.reshape(1, D),
  )
  return out_t.transpose(2, 0, 1)


# speedup vs baseline: 1.0256x; 1.0256x over previous
import functools

import jax
import jax.numpy as jnp
from jax import lax
from jax.experimental import pallas as pl
from jax.experimental.pallas import tpu as pltpu
from jax.experimental.pallas import tpu_sc as plsc

NUM_ARMS = 1000000
D = 32
B = 4096
S = 50
N = B * S

NC = 2
NS = 16
NW = NC * NS
CHUNK = B // NW


def _sc_gather_body(action_hbm, table_hbm, out_hbm, idx_v, buf0, buf1,
                    sem0, sem1):
  wid = lax.axis_index("s") * NC + lax.axis_index("c")
  bbase = wid * CHUNK
  pltpu.sync_copy(action_hbm.at[:, pl.ds(bbase, CHUNK)], idx_v)

  bufs = (buf0, buf1)
  sems = (sem0, sem1)

  pltpu.async_copy(table_hbm.at[idx_v.at[0]], buf0, sem0)
  pltpu.async_copy(table_hbm.at[idx_v.at[1]], buf1, sem1)

  def step(i, _):
    base = i * 2
    for b in range(2):
      s = base + b
      pltpu.make_async_copy(table_hbm.at[idx_v.at[s]], bufs[b], sems[b]).wait()
      pltpu.sync_copy(bufs[b], out_hbm.at[pl.ds(s * B + bbase, CHUNK)])
      @pl.when(s + 2 < S)
      def _():
        pltpu.async_copy(table_hbm.at[idx_v.at[s + 2]], bufs[b], sems[b])
    return 0

  lax.fori_loop(0, S // 2, step, 0)


_sc_gather = functools.partial(
    pl.kernel,
    out_type=jax.ShapeDtypeStruct((N, D), jnp.float32),
    mesh=plsc.VectorSubcoreMesh(core_axis_name="c", subcore_axis_name="s"),
    scratch_types=[
        pltpu.VMEM((S, CHUNK), jnp.int32),
        pltpu.VMEM((CHUNK, D), jnp.float32),
        pltpu.VMEM((CHUNK, D), jnp.float32),
        pltpu.SemaphoreType.DMA,
        pltpu.SemaphoreType.DMA,
    ],
    compiler_params=pltpu.CompilerParams(use_tc_tiling_on_sc=False),
)(_sc_gather_body)


TBB = 1024
NJ = B // TBB


def _tc_assemble_body(state_ref, reward_ref, act_hbm, eye_ref, wo_ref,
                      bo_ref, wr_ref, br_ref, out_ref, abuf0, abuf1,
                      asem0, asem1):
  s = pl.program_id(0)
  j = pl.program_id(1)
  step = s * NJ + j
  abufs = (abuf0, abuf1)
  asems = (asem0, asem1)

  def act_copy(st, b):
    tok0 = (st // NJ) * B + (st % NJ) * TBB
    return pltpu.make_async_copy(
        act_hbm.at[pl.ds(tok0, TBB), :], abufs[b], asems[b])

  @pl.when(step == 0)
  def _():
    act_copy(0, 0).start()

  nxt = step + 1
  for b in range(2):
    @pl.when((nxt < S * NJ) & (nxt % 2 == b))
    def _(b=b):
      act_copy(nxt, b).start()

  for b in range(2):
    @pl.when(step % 2 == b)
    def _(b=b):
      act_copy(step, b).wait()
  act_blk = jnp.where(step % 2 == 0, abuf0[...], abuf1[...])
  act_t = lax.dot_general(
      eye_ref[...], act_blk, (((1,), (1,)), ((), ())),
      preferred_element_type=jnp.float32,
      precision=lax.Precision.HIGHEST)

  wo = jnp.transpose(wo_ref[...])
  bo = jnp.transpose(bo_ref[...])
  wr = jnp.transpose(wr_ref[...])
  br = jnp.transpose(br_ref[...])
  st = state_ref[0]
  rw = reward_ref[0]
  out_ref[0] = wo * st + bo
  out_ref[1] = act_t
  out_ref[2] = wr * rw + br


def _tc_assemble(state_t, reward_t, act_c, eye, W_obs, b_obs, W_rew, b_rew):
  grid = (S, NJ)
  return pl.pallas_call(
      _tc_assemble_body,
      grid=grid,
      in_specs=[
          pl.BlockSpec((1, 1, TBB), lambda s, j: (s, 0, j)),
          pl.BlockSpec((1, 1, TBB), lambda s, j: (s, 0, j)),
          pl.BlockSpec(memory_space=pl.ANY),
          pl.BlockSpec((D, D), lambda s, j: (0, 0)),
          pl.BlockSpec((1, D), lambda s, j: (0, 0)),
          pl.BlockSpec((1, D), lambda s, j: (0, 0)),
          pl.BlockSpec((1, D), lambda s, j: (0, 0)),
          pl.BlockSpec((1, D), lambda s, j: (0, 0)),
      ],
      out_specs=pl.BlockSpec((3, D, TBB), lambda s, j: (s, 0, j)),
      out_shape=jax.ShapeDtypeStruct((3 * S, D, B), jnp.float32),
      scratch_shapes=[
          pltpu.VMEM((TBB, D), jnp.float32),
          pltpu.VMEM((TBB, D), jnp.float32),
          pltpu.SemaphoreType.DMA,
          pltpu.SemaphoreType.DMA,
      ],
  )(state_t, reward_t, act_c, eye, W_obs, b_obs, W_rew, b_rew)


@jax.jit
def kernel(state, action, reward, W_obs, b_obs, emb_table, W_rew, b_rew):
  action_t = action.astype(jnp.int32).T
  state_t = state.transpose(1, 2, 0)
  reward_t = reward.T.reshape(S, 1, B)
  act_c = _sc_gather(action_t, emb_table)
  eye = jnp.eye(D, dtype=jnp.float32)
  out_t = _tc_assemble(
      state_t,
      reward_t,
      act_c,
      eye,
      W_obs,
      b_obs.reshape(1, D),
      W_rew,
      b_rew.reshape(1, D),
  )
  return out_t.transpose(2, 0, 1)


# speedup vs baseline: 1.0814x; 1.0544x over previous
import functools

import jax
import jax.numpy as jnp
from jax import lax
from jax.experimental import pallas as pl
from jax.experimental.pallas import tpu as pltpu
from jax.experimental.pallas import tpu_sc as plsc

NUM_ARMS = 1000000
D = 32
B = 4096
S = 50
N = B * S
DP = 128

NC = 2
NS = 16
NW = NC * NS
CHUNK = B // NW


def _sc_gather_body(action_hbm, table_hbm, out_hbm, idx_v, buf0, buf1,
                    sem0, sem1):
  wid = lax.axis_index("s") * NC + lax.axis_index("c")
  bbase = wid * CHUNK
  pltpu.sync_copy(action_hbm.at[:, pl.ds(bbase, CHUNK)], idx_v)

  bufs = (buf0, buf1)
  sems = (sem0, sem1)

  pltpu.async_copy(table_hbm.at[idx_v.at[0]], buf0, sem0)
  pltpu.async_copy(table_hbm.at[idx_v.at[1]], buf1, sem1)

  def step(i, _):
    base = i * 2
    for b in range(2):
      s = base + b
      pltpu.make_async_copy(table_hbm.at[idx_v.at[s]], bufs[b], sems[b]).wait()
      pltpu.sync_copy(bufs[b], out_hbm.at[pl.ds(s * B + bbase, CHUNK)])
      @pl.when(s + 2 < S)
      def _():
        pltpu.async_copy(table_hbm.at[idx_v.at[s + 2]], bufs[b], sems[b])
    return 0

  lax.fori_loop(0, S // 2, step, 0)


_sc_gather = functools.partial(
    pl.kernel,
    out_type=jax.ShapeDtypeStruct((N, DP), jnp.float32),
    mesh=plsc.VectorSubcoreMesh(core_axis_name="c", subcore_axis_name="s"),
    scratch_types=[
        pltpu.VMEM((S, CHUNK), jnp.int32),
        pltpu.VMEM((CHUNK, DP), jnp.float32),
        pltpu.VMEM((CHUNK, DP), jnp.float32),
        pltpu.SemaphoreType.DMA,
        pltpu.SemaphoreType.DMA,
    ],
    compiler_params=pltpu.CompilerParams(use_tc_tiling_on_sc=True,
                                         needs_layout_passes=False),
)(_sc_gather_body)


TBB = 1024


def _tc_assemble_body(state_ref, reward_ref, act_ref, eye_ref, wo_ref,
                      bo_ref, wr_ref, br_ref, out_ref):
  out_ref[1] = lax.dot_general(
      eye_ref[...], act_ref[0], (((1,), (1,)), ((), ())),
      preferred_element_type=jnp.float32,
      precision=lax.Precision.HIGHEST)
  wo = jnp.transpose(wo_ref[...])
  bo = jnp.transpose(bo_ref[...])
  wr = jnp.transpose(wr_ref[...])
  br = jnp.transpose(br_ref[...])
  st = state_ref[0]
  rw = reward_ref[0]
  out_ref[0] = wo * st + bo
  out_ref[2] = wr * rw + br


def _tc_assemble(state_t, reward_t, act_c, eye, W_obs, b_obs, W_rew, b_rew):
  grid = (S, B // TBB)
  return pl.pallas_call(
      _tc_assemble_body,
      grid=grid,
      in_specs=[
          pl.BlockSpec((1, 1, TBB), lambda s, j: (s, 0, j)),
          pl.BlockSpec((1, 1, TBB), lambda s, j: (s, 0, j)),
          pl.BlockSpec((1, TBB, DP), lambda s, j: (s, j, 0)),
          pl.BlockSpec((D, DP), lambda s, j: (0, 0)),
          pl.BlockSpec((1, D), lambda s, j: (0, 0)),
          pl.BlockSpec((1, D), lambda s, j: (0, 0)),
          pl.BlockSpec((1, D), lambda s, j: (0, 0)),
          pl.BlockSpec((1, D), lambda s, j: (0, 0)),
      ],
      out_specs=pl.BlockSpec((3, D, TBB), lambda s, j: (s, 0, j)),
      out_shape=jax.ShapeDtypeStruct((3 * S, D, B), jnp.float32),
  )(state_t, reward_t, act_c, eye, W_obs, b_obs, W_rew, b_rew)


@jax.jit
def kernel(state, action, reward, W_obs, b_obs, emb_table, W_rew, b_rew):
  action_t = action.astype(jnp.int32).T
  state_t = state.transpose(1, 2, 0)
  reward_t = reward.T.reshape(S, 1, B)
  table_p = jnp.pad(emb_table, ((0, 0), (0, DP - D)))
  act_c = _sc_gather(action_t, table_p)
  eye = jnp.eye(D, DP, dtype=jnp.float32)
  out_t = _tc_assemble(
      state_t,
      reward_t,
      act_c.reshape(S, B, DP),
      eye,
      W_obs,
      b_obs.reshape(1, D),
      W_rew,
      b_rew.reshape(1, D),
  )
  return out_t.transpose(2, 0, 1)
